# baseline (speedup 1.0000x reference)
import functools

import jax
import jax.numpy as jnp
import numpy as np
from jax import lax
from jax.experimental import pallas as pl
from jax.experimental.pallas import tpu as pltpu
from jax.experimental.pallas import tpu_sc as plsc

_ANCHORS = np.array([[10.0, 13.0], [16.0, 30.0], [33.0, 23.0]], dtype=np.float32)
_IMG_DIM = 608.0
_NA = 3
_NW = 32
_X0S = (0, 16, 32, 48, 60)


def kernel(x):
    nB, C, nG, _ = x.shape
    attrs = C // _NA
    S = nG * nG
    stride = _IMG_DIM / nG
    RPS = 4
    n_strips = nG // RPS
    SPW = RPS * nG
    n_tasks = nB * _NA * n_strips
    n_per = -(-n_tasks // _NW)

    mesh = plsc.VectorSubcoreMesh(core_axis_name="c", subcore_axis_name="s")

    @functools.partial(
        pl.kernel,
        out_type=jax.ShapeDtypeStruct((nB, _NA * S, attrs), jnp.float32),
        mesh=mesh,
        scratch_types=[
            pltpu.VMEM((attrs, RPS, nG), jnp.float32),
            pltpu.VMEM((SPW, attrs), jnp.float32),
        ],
        compiler_params=pltpu.CompilerParams(use_tc_tiling_on_sc=True, needs_layout_passes=False),
    )
    def sc_k(x_hbm, out_hbm, in_v, out_v):
        wid = lax.axis_index("s") * 2 + lax.axis_index("c")
        iota = lax.iota(jnp.int32, 16)
        iotaf = iota.astype(jnp.float32)

        def task_body(i, carry):
            t = wid + _NW * i

            @pl.when(t < n_tasks)
            def _():
                b = t // (_NA * n_strips)
                r1 = t % (_NA * n_strips)
                a = r1 // n_strips
                st = r1 % n_strips
                y0 = st * RPS
                c0 = a * attrs
                row0 = a * S + st * SPW

                pltpu.sync_copy(x_hbm.at[b, pl.ds(c0, attrs), pl.ds(y0, RPS), :], in_v)

                aw = jnp.where(
                    a == 0, _ANCHORS[0, 0], jnp.where(a == 1, _ANCHORS[1, 0], _ANCHORS[2, 0])
                )
                ah = jnp.where(
                    a == 0, _ANCHORS[0, 1], jnp.where(a == 1, _ANCHORS[1, 1], _ANCHORS[2, 1])
                )
                y0f = y0.astype(jnp.float32)

                for c in range(4):
                    for r in range(RPS):
                        for x0 in _X0S:
                            v = in_v[c, r, pl.ds(x0, 16)]
                            if c == 0:
                                res = (1.0 / (1.0 + jnp.exp(-v)) + (x0 + iotaf)) * stride
                            elif c == 1:
                                gy = y0f + float(r)
                                res = (1.0 / (1.0 + jnp.exp(-v)) + gy) * stride
                            elif c == 2:
                                res = jnp.exp(v) * aw
                            else:
                                res = jnp.exp(v) * ah
                            rows = (r * nG + x0) + iota
                            cols = jnp.full((16,), c, dtype=jnp.int32)
                            plsc.store_scatter(out_v, [rows, cols], res)

                def cls_body(c, carry2):
                    for r in range(RPS):
                        for x0 in _X0S:
                            v = in_v[c, r, pl.ds(x0, 16)]
                            res = 1.0 / (1.0 + jnp.exp(-v))
                            rows = (r * nG + x0) + iota
                            cols = jnp.full((16,), 0, dtype=jnp.int32) + c
                            plsc.store_scatter(out_v, [rows, cols], res)
                    return carry2

                lax.fori_loop(4, attrs, cls_body, 0)

                pltpu.sync_copy(out_v, out_hbm.at[b, pl.ds(row0, SPW), :])

            return carry

        lax.fori_loop(0, n_per, task_body, 0)

    return sc_k(x)


# speedup vs baseline: 2.0322x; 2.0322x over previous
import functools

import jax
import jax.numpy as jnp
import numpy as np
from jax import lax
from jax.experimental import pallas as pl
from jax.experimental.pallas import tpu as pltpu
from jax.experimental.pallas import tpu_sc as plsc

_ANCHORS = np.array([[10.0, 13.0], [16.0, 30.0], [33.0, 23.0]], dtype=np.float32)
_IMG_DIM = 608.0
_NA = 3
_NW = 32
_X0S = (0, 16, 32, 48, 60)


def kernel(x):
    nB, C, nG, _ = x.shape
    attrs = C // _NA
    S = nG * nG
    stride = _IMG_DIM / nG
    RPS = 4
    n_strips = nG // RPS
    SPW = RPS * nG
    n_tasks = nB * _NA * n_strips
    n_per = -(-n_tasks // _NW)

    mesh = plsc.VectorSubcoreMesh(core_axis_name="c", subcore_axis_name="s")

    @functools.partial(
        pl.kernel,
        out_type=jax.ShapeDtypeStruct((nB, _NA * S, attrs), jnp.float32),
        mesh=mesh,
        scratch_types=[
            pltpu.VMEM((attrs, RPS, nG), jnp.float32),
            pltpu.VMEM((SPW, attrs), jnp.float32),
        ],
        compiler_params=pltpu.CompilerParams(use_tc_tiling_on_sc=True, needs_layout_passes=False),
    )
    def sc_k(x_hbm, out_hbm, in_v, out_v):
        wid = lax.axis_index("s") * 2 + lax.axis_index("c")
        iota = lax.iota(jnp.int32, 16)
        iotaf = iota.astype(jnp.float32)

        def task_body(i, carry):
            t = wid + _NW * i

            @pl.when(t < n_tasks)
            def _():
                b = t // (_NA * n_strips)
                r1 = t % (_NA * n_strips)
                a = r1 // n_strips
                st = r1 % n_strips
                y0 = st * RPS
                c0 = a * attrs
                row0 = a * S + st * SPW

                pltpu.sync_copy(x_hbm.at[b, pl.ds(c0, attrs), pl.ds(y0, RPS), :], in_v)

                aw = jnp.where(
                    a == 0, _ANCHORS[0, 0], jnp.where(a == 1, _ANCHORS[1, 0], _ANCHORS[2, 0])
                )
                ah = jnp.where(
                    a == 0, _ANCHORS[0, 1], jnp.where(a == 1, _ANCHORS[1, 1], _ANCHORS[2, 1])
                )
                y0f = y0.astype(jnp.float32)

                for c in range(4):
                    for r in range(RPS):
                        for x0 in _X0S:
                            v = in_v[c, r, pl.ds(x0, 16)]
                            if c == 0:
                                res = (1.0 / (1.0 + jnp.exp(-v)) + (x0 + iotaf)) * stride
                            elif c == 1:
                                gy = y0f + float(r)
                                res = (1.0 / (1.0 + jnp.exp(-v)) + gy) * stride
                            elif c == 2:
                                res = jnp.exp(v) * aw
                            else:
                                res = jnp.exp(v) * ah
                            rows = (r * nG + x0) + iota
                            cols = jnp.full((16,), c, dtype=jnp.int32)
                            plsc.store_scatter(out_v, [rows, cols], res)

                @plsc.parallel_loop(4, attrs, step=1, unroll=2)
                def cls_body(c):
                    for r in range(RPS):
                        for x0 in _X0S:
                            v = in_v[c, r, pl.ds(x0, 16)]
                            res = 1.0 / (1.0 + jnp.exp(-v))
                            rows = (r * nG + x0) + iota
                            cols = jnp.full((16,), 0, dtype=jnp.int32) + c
                            plsc.store_scatter(out_v, [rows, cols], res)

                pltpu.sync_copy(out_v, out_hbm.at[b, pl.ds(row0, SPW), :])

            return carry

        lax.fori_loop(0, n_per, task_body, 0)

    return sc_k(x)
